# Initial kernel scaffold; baseline (speedup 1.0000x reference)
import jax
import jax.numpy as jnp
from jax.experimental import pallas as pl


def kernel(x, dijk, idx_j, seg_i, seg_j, W1, b1, W2, b2, Wf, bf, Wo, bo, Wd, bd):
    raise NotImplementedError("write your pallas kernel here")


# speedup vs baseline: 3.6674x; 3.6674x over previous
import functools
import math

import jax
import jax.numpy as jnp
from jax import lax
from jax.experimental import pallas as pl
from jax.experimental.pallas import tpu as pltpu
from jax.experimental.pallas import tpu_sc as plsc

_LOG2 = math.log(2.0)


def _ssp(t):
    return jnp.maximum(t, 0.0) + jnp.log1p(jnp.exp(-jnp.abs(t))) - _LOG2


def _f_body(x_ref, wf_ref, bf_ref, o_ref):
    o_ref[...] = (
        jnp.dot(x_ref[...], wf_ref[...], preferred_element_type=jnp.float32)
        + bf_ref[...]
    )


def _edge_body(dijk_ref, fjj_ref, w1_ref, b1_ref, w2_ref, b2_ref, o_ref):
    h = _ssp(jnp.dot(dijk_ref[...], w1_ref[...],
                     preferred_element_type=jnp.float32) + b1_ref[...])
    w = _ssp(jnp.dot(h, w2_ref[...],
                     preferred_element_type=jnp.float32) + b2_ref[...])
    o_ref[...] = w * fjj_ref[...]


def _out_body(conv2_ref, x_ref, wo_ref, bo_ref, wd_ref, bd_ref, y_ref, v_ref):
    conv = conv2_ref[0] + conv2_ref[1]
    h2 = _ssp(jnp.dot(conv, wo_ref[...],
                      preferred_element_type=jnp.float32) + bo_ref[...])
    v = jnp.dot(h2, wd_ref[...], preferred_element_type=jnp.float32) + bd_ref[...]
    v_ref[...] = v
    y_ref[...] = x_ref[...] + v


_NC = 2
_NS = 16
_NW = _NC * _NS
_CG = 512
_G = _CG // 128
_CS = 256
_GS = _CS // 128


def _sc_gather(f_hbm, jj_hbm, o_hbm, fbuf, jbuf, sem):
    nch = o_hbm.shape[0] // _CG
    k_max = (nch + _NW - 1) // _NW
    wid = lax.axis_index("s") * _NC + lax.axis_index("c")

    def chunk(ci):
        pltpu.sync_copy(jj_hbm.at[pl.ds(ci * _G, _G)], jbuf)
        cps = [
            pltpu.async_copy(f_hbm.at[jbuf.at[g]],
                             fbuf.at[pl.ds(g * 128, 128)], sem)
            for g in range(_G)
        ]
        for cp in cps:
            cp.wait()
        pltpu.sync_copy(fbuf, o_hbm.at[pl.ds(ci * _CG, _CG)])

    def body(k, carry):
        ci = wid + k * _NW

        @pl.when(ci < nch)
        def _():
            chunk(ci)

        return carry

    lax.fori_loop(0, k_max, body, 0)


def _sc_scatter(wf_hbm, ii_hbm, z_hbm, o_hbm, wbuf, ibuf, acc, sem):
    n_pad = o_hbm.shape[1]
    e_rows = wf_hbm.shape[0]
    nch = e_rows // _CS
    k_max = (nch + _NW - 1) // _NW
    cid = lax.axis_index("c")
    sid = lax.axis_index("s")
    wid = sid * _NC + cid
    rows = n_pad // _NS

    def inner():
        pltpu.sync_copy(z_hbm.at[pl.ds(sid * rows, rows)],
                        acc.at[pl.ds(sid * rows, rows)])
        plsc.subcore_barrier()

        def chunk(ci):
            pltpu.sync_copy(wf_hbm.at[pl.ds(ci * _CS, _CS)], wbuf)
            pltpu.sync_copy(ii_hbm.at[pl.ds(ci * _GS, _GS)], ibuf)
            for g in range(_GS):
                pltpu.sync_copy(wbuf.at[pl.ds(g * 128, 128)],
                                acc.at[ibuf.at[g]], add=True)

        def body(k, carry):
            ci = wid + k * _NW

            @pl.when(ci < nch)
            def _():
                chunk(ci)

            return carry

        lax.fori_loop(0, k_max, body, 0)
        plsc.subcore_barrier()
        pltpu.sync_copy(acc.at[pl.ds(sid * rows, rows)],
                        o_hbm.at[cid, pl.ds(sid * rows, rows)])

    inner()


def kernel(x, dijk, idx_j, seg_i, seg_j, W1, b1, W2, b2, Wf, bf, Wo, bo, Wd, bd):
    n_nodes, d = x.shape
    e_rows = dijk.shape[0]

    jj = jnp.take(idx_j, seg_j)
    ii = jnp.take(seg_i, seg_j)
    jj2 = jj.reshape(e_rows // 128, 128)
    ii2 = ii.reshape(e_rows // 128, 128)
    n_pad = ((n_nodes + (8 * _NS) - 1) // (8 * _NS)) * (8 * _NS)
    zeros = jnp.zeros((n_pad, d), jnp.float32)

    bn = 2000
    f = pl.pallas_call(
        _f_body,
        grid=(n_nodes // bn,),
        in_specs=[
            pl.BlockSpec((bn, d), lambda i: (i, 0)),
            pl.BlockSpec((d, d), lambda i: (0, 0)),
            pl.BlockSpec((1, d), lambda i: (0, 0)),
        ],
        out_specs=pl.BlockSpec((bn, d), lambda i: (i, 0)),
        out_shape=jax.ShapeDtypeStruct((n_nodes, d), jnp.float32),
    )(x, Wf, bf.reshape(1, d))

    mesh = plsc.VectorSubcoreMesh(core_axis_name="c", subcore_axis_name="s")
    fjj = pl.kernel(
        _sc_gather,
        out_type=jax.ShapeDtypeStruct((e_rows, d), jnp.float32),
        mesh=mesh,
        scratch_types=[
            pltpu.VMEM((_CG, d), jnp.float32),
            pltpu.VMEM((_G, 128), jnp.int32),
            pltpu.SemaphoreType.DMA,
        ],
    )(f, jj2)

    be = 2560
    wf = pl.pallas_call(
        _edge_body,
        grid=(e_rows // be,),
        in_specs=[
            pl.BlockSpec((be, d), lambda i: (i, 0)),
            pl.BlockSpec((be, d), lambda i: (i, 0)),
            pl.BlockSpec((d, d), lambda i: (0, 0)),
            pl.BlockSpec((1, d), lambda i: (0, 0)),
            pl.BlockSpec((d, d), lambda i: (0, 0)),
            pl.BlockSpec((1, d), lambda i: (0, 0)),
        ],
        out_specs=pl.BlockSpec((be, d), lambda i: (i, 0)),
        out_shape=jax.ShapeDtypeStruct((e_rows, d), jnp.float32),
    )(dijk, fjj, W1, b1.reshape(1, d), W2, b2.reshape(1, d))

    conv2 = pl.kernel(
        _sc_scatter,
        out_type=jax.ShapeDtypeStruct((_NC, n_pad, d), jnp.float32),
        mesh=mesh,
        scratch_types=[
            pltpu.VMEM((_CS, d), jnp.float32),
            pltpu.VMEM((_GS, 128), jnp.int32),
            pltpu.VMEM_SHARED((n_pad, d), jnp.float32),
            pltpu.SemaphoreType.DMA,
        ],
    )(wf, ii2, zeros)

    y, v = pl.pallas_call(
        _out_body,
        grid=(n_nodes // bn,),
        in_specs=[
            pl.BlockSpec((_NC, bn, d), lambda i: (0, i, 0)),
            pl.BlockSpec((bn, d), lambda i: (i, 0)),
            pl.BlockSpec((d, d), lambda i: (0, 0)),
            pl.BlockSpec((1, d), lambda i: (0, 0)),
            pl.BlockSpec((d, d), lambda i: (0, 0)),
            pl.BlockSpec((1, d), lambda i: (0, 0)),
        ],
        out_specs=[
            pl.BlockSpec((bn, d), lambda i: (i, 0)),
            pl.BlockSpec((bn, d), lambda i: (i, 0)),
        ],
        out_shape=[
            jax.ShapeDtypeStruct((n_nodes, d), jnp.float32),
            jax.ShapeDtypeStruct((n_nodes, d), jnp.float32),
        ],
    )(conv2, x, Wo, bo.reshape(1, d), Wd, bd.reshape(1, d))
    return (y, v)


# speedup vs baseline: 4.3040x; 1.1736x over previous
import functools
import math

import jax
import jax.numpy as jnp
from jax import lax
from jax.experimental import pallas as pl
from jax.experimental.pallas import tpu as pltpu
from jax.experimental.pallas import tpu_sc as plsc

_LOG2 = math.log(2.0)


def _ssp(t):
    return jnp.where(t > 60.0, t - _LOG2, jnp.log(0.5 * jnp.exp(t) + 0.5))


def _f_body(x_ref, wf_ref, bf_ref, o_ref):
    o_ref[...] = (
        jnp.dot(x_ref[...], wf_ref[...], preferred_element_type=jnp.float32)
        + bf_ref[...]
    )


def _edge_body(dijk_ref, fjj_ref, w1_ref, b1_ref, w2_ref, b2_ref, o_ref):
    h = _ssp(jnp.dot(dijk_ref[...], w1_ref[...],
                     preferred_element_type=jnp.float32) + b1_ref[...])
    w = _ssp(jnp.dot(h, w2_ref[...],
                     preferred_element_type=jnp.float32) + b2_ref[...])
    o_ref[...] = w * fjj_ref[...]


def _out_body(conv2_ref, x_ref, wo_ref, bo_ref, wd_ref, bd_ref, y_ref, v_ref):
    conv = conv2_ref[0] + conv2_ref[1]
    h2 = _ssp(jnp.dot(conv, wo_ref[...],
                      preferred_element_type=jnp.float32) + bo_ref[...])
    v = jnp.dot(h2, wd_ref[...], preferred_element_type=jnp.float32) + bd_ref[...]
    v_ref[...] = v
    y_ref[...] = x_ref[...] + v


_NC = 2
_NS = 16
_NW = _NC * _NS
_CG = 256
_G = _CG // 128
_CS = 128
_GS = _CS // 128


def _drain(dummy_hbm, dst, sem):
    pltpu.make_async_copy(dummy_hbm, dst, sem).wait()


def _sc_gather(f_hbm, jj_hbm, o_hbm,
               fbuf0, fbuf1, jbuf0, jbuf1, sg0, sg1, so0, so1):
    nch = o_hbm.shape[0] // _CG
    k_max = (nch + 2 * _NW - 1) // (2 * _NW) * 2
    wid = lax.axis_index("s") * _NC + lax.axis_index("c")
    bufs = ((fbuf0, jbuf0, sg0, so0), (fbuf1, jbuf1, sg1, so1))
    dummy = f_hbm.at[pl.ds(0, _CG)]

    def fire(k, b):
        fbuf, jbuf, sg, _ = bufs[b]
        ci = wid + k * _NW

        @pl.when((ci >= 0) & (ci < nch))
        def _():
            pltpu.sync_copy(jj_hbm.at[pl.ds(ci * _G, _G)], jbuf)
            for g in range(_G):
                pltpu.async_copy(f_hbm.at[jbuf.at[g]],
                                 fbuf.at[pl.ds(g * 128, 128)], sg)

    def finish(k, b):
        fbuf, _, sg, so = bufs[b]
        ci = wid + k * _NW

        @pl.when((ci >= 0) & (ci < nch))
        def _():
            _drain(dummy, fbuf, sg)
            pltpu.async_copy(fbuf, o_hbm.at[pl.ds(ci * _CG, _CG)], so)

    def drain_out(k, b):
        fbuf, _, _, so = bufs[b]
        ci = wid + k * _NW

        @pl.when((ci >= 0) & (ci < nch))
        def _():
            _drain(dummy, fbuf, so)

    fire(0, 0)

    def body(k2, carry):
        for b in range(2):
            k = 2 * k2 + b
            nb = 1 - b
            drain_out(k - 1, nb)
            fire(k + 1, nb)
            finish(k, b)
        return carry

    lax.fori_loop(0, k_max // 2, body, 0)
    drain_out(k_max - 1, (k_max - 1) % 2)
    drain_out(k_max, k_max % 2)


def _sc_scatter(wf_hbm, ii_hbm, z_hbm, o_hbm,
                wbuf0, wbuf1, ibuf0, ibuf1, si0, si1, ss0, ss1, acc):
    n_pad = o_hbm.shape[1]
    e_rows = wf_hbm.shape[0]
    nch = e_rows // _CS
    k_max = (nch + 2 * _NW - 1) // (2 * _NW) * 2
    cid = lax.axis_index("c")
    sid = lax.axis_index("s")
    wid = sid * _NC + cid
    rows = n_pad // _NS
    bufs = ((wbuf0, ibuf0, si0, ss0), (wbuf1, ibuf1, si1, ss1))
    dummy = wf_hbm.at[pl.ds(0, _CS)]

    pltpu.sync_copy(z_hbm.at[pl.ds(sid * rows, rows)],
                    acc.at[pl.ds(sid * rows, rows)])
    plsc.subcore_barrier()

    def fire(k, b):
        wbuf, ibuf, si, _ = bufs[b]
        ci = wid + k * _NW

        @pl.when((ci >= 0) & (ci < nch))
        def _():
            pltpu.sync_copy(ii_hbm.at[pl.ds(ci * _GS, _GS)], ibuf)
            pltpu.async_copy(wf_hbm.at[pl.ds(ci * _CS, _CS)], wbuf, si)

    def finish(k, b):
        wbuf, ibuf, si, ss = bufs[b]
        ci = wid + k * _NW

        @pl.when((ci >= 0) & (ci < nch))
        def _():
            _drain(dummy, wbuf, si)
            for g in range(_GS):
                pltpu.async_copy(wbuf.at[pl.ds(g * 128, 128)],
                                 acc.at[ibuf.at[g]], ss, add=True)

    def drain_sc(k, b):
        wbuf, _, _, ss = bufs[b]
        ci = wid + k * _NW

        @pl.when((ci >= 0) & (ci < nch))
        def _():
            _drain(dummy, wbuf, ss)

    fire(0, 0)

    def body(k2, carry):
        for b in range(2):
            k = 2 * k2 + b
            nb = 1 - b
            drain_sc(k - 1, nb)
            fire(k + 1, nb)
            finish(k, b)
        return carry

    lax.fori_loop(0, k_max // 2, body, 0)
    drain_sc(k_max - 1, (k_max - 1) % 2)
    drain_sc(k_max, k_max % 2)
    plsc.subcore_barrier()
    pltpu.sync_copy(acc.at[pl.ds(sid * rows, rows)],
                    o_hbm.at[cid, pl.ds(sid * rows, rows)])


def kernel(x, dijk, idx_j, seg_i, seg_j, W1, b1, W2, b2, Wf, bf, Wo, bo, Wd, bd):
    n_nodes, d = x.shape
    e_rows = dijk.shape[0]

    jj = jnp.take(idx_j, seg_j)
    ii = jnp.take(seg_i, seg_j)
    jj2 = jj.reshape(e_rows // 128, 128)
    ii2 = ii.reshape(e_rows // 128, 128)
    n_pad = ((n_nodes + (8 * _NS) - 1) // (8 * _NS)) * (8 * _NS)
    zeros = jnp.zeros((n_pad, d), jnp.float32)

    bn = 2000
    f = pl.pallas_call(
        _f_body,
        grid=(n_nodes // bn,),
        in_specs=[
            pl.BlockSpec((bn, d), lambda i: (i, 0)),
            pl.BlockSpec((d, d), lambda i: (0, 0)),
            pl.BlockSpec((1, d), lambda i: (0, 0)),
        ],
        out_specs=pl.BlockSpec((bn, d), lambda i: (i, 0)),
        out_shape=jax.ShapeDtypeStruct((n_nodes, d), jnp.float32),
    )(x, Wf, bf.reshape(1, d))

    mesh = plsc.VectorSubcoreMesh(core_axis_name="c", subcore_axis_name="s")
    fjj = pl.kernel(
        _sc_gather,
        out_type=jax.ShapeDtypeStruct((e_rows, d), jnp.float32),
        mesh=mesh,
        scratch_types=[
            pltpu.VMEM((_CG, d), jnp.float32),
            pltpu.VMEM((_CG, d), jnp.float32),
            pltpu.VMEM((_G, 128), jnp.int32),
            pltpu.VMEM((_G, 128), jnp.int32),
            pltpu.SemaphoreType.DMA,
            pltpu.SemaphoreType.DMA,
            pltpu.SemaphoreType.DMA,
            pltpu.SemaphoreType.DMA,
        ],
    )(f, jj2)

    be = 2560
    wf = pl.pallas_call(
        _edge_body,
        grid=(e_rows // be,),
        in_specs=[
            pl.BlockSpec((be, d), lambda i: (i, 0)),
            pl.BlockSpec((be, d), lambda i: (i, 0)),
            pl.BlockSpec((d, d), lambda i: (0, 0)),
            pl.BlockSpec((1, d), lambda i: (0, 0)),
            pl.BlockSpec((d, d), lambda i: (0, 0)),
            pl.BlockSpec((1, d), lambda i: (0, 0)),
        ],
        out_specs=pl.BlockSpec((be, d), lambda i: (i, 0)),
        out_shape=jax.ShapeDtypeStruct((e_rows, d), jnp.float32),
    )(dijk, fjj, W1, b1.reshape(1, d), W2, b2.reshape(1, d))

    conv2 = pl.kernel(
        _sc_scatter,
        out_type=jax.ShapeDtypeStruct((_NC, n_pad, d), jnp.float32),
        mesh=mesh,
        scratch_types=[
            pltpu.VMEM((_CS, d), jnp.float32),
            pltpu.VMEM((_CS, d), jnp.float32),
            pltpu.VMEM((_GS, 128), jnp.int32),
            pltpu.VMEM((_GS, 128), jnp.int32),
            pltpu.SemaphoreType.DMA,
            pltpu.SemaphoreType.DMA,
            pltpu.SemaphoreType.DMA,
            pltpu.SemaphoreType.DMA,
            pltpu.VMEM_SHARED((n_pad, d), jnp.float32),
        ],
    )(wf, ii2, zeros)

    y, v = pl.pallas_call(
        _out_body,
        grid=(n_nodes // bn,),
        in_specs=[
            pl.BlockSpec((_NC, bn, d), lambda i: (0, i, 0)),
            pl.BlockSpec((bn, d), lambda i: (i, 0)),
            pl.BlockSpec((d, d), lambda i: (0, 0)),
            pl.BlockSpec((1, d), lambda i: (0, 0)),
            pl.BlockSpec((d, d), lambda i: (0, 0)),
            pl.BlockSpec((1, d), lambda i: (0, 0)),
        ],
        out_specs=[
            pl.BlockSpec((bn, d), lambda i: (i, 0)),
            pl.BlockSpec((bn, d), lambda i: (i, 0)),
        ],
        out_shape=[
            jax.ShapeDtypeStruct((n_nodes, d), jnp.float32),
            jax.ShapeDtypeStruct((n_nodes, d), jnp.float32),
        ],
    )(conv2, x, Wo, bo.reshape(1, d), Wd, bd.reshape(1, d))
    return (y, v)


# speedup vs baseline: 4.5931x; 1.0672x over previous
import functools
import math

import jax
import jax.numpy as jnp
from jax import lax
from jax.experimental import pallas as pl
from jax.experimental.pallas import tpu as pltpu
from jax.experimental.pallas import tpu_sc as plsc

_LOG2 = math.log(2.0)


def _ssp(t):
    return jnp.where(t > 60.0, t - _LOG2, jnp.log(0.5 * jnp.exp(t) + 0.5))


def _f_body(x_ref, wf_ref, bf_ref, o_ref):
    o_ref[...] = (
        jnp.dot(x_ref[...], wf_ref[...], preferred_element_type=jnp.float32)
        + bf_ref[...]
    )


def _edge_body(dijk_ref, fjj_ref, w1_ref, b1_ref, w2_ref, b2_ref, o_ref):
    h = _ssp(jnp.dot(dijk_ref[...], w1_ref[...],
                     preferred_element_type=jnp.float32) + b1_ref[...])
    w = _ssp(jnp.dot(h, w2_ref[...],
                     preferred_element_type=jnp.float32) + b2_ref[...])
    o_ref[...] = w * fjj_ref[...]


def _out_body(*refs):
    conv_refs = refs[:-7]
    x_ref, wo_ref, bo_ref, wd_ref, bd_ref, y_ref, v_ref = refs[-7:]
    conv = conv_refs[0][0] + conv_refs[0][1]
    for c_ref in conv_refs[1:]:
        conv = conv + c_ref[0] + c_ref[1]
    h2 = _ssp(jnp.dot(conv, wo_ref[...],
                      preferred_element_type=jnp.float32) + bo_ref[...])
    v = jnp.dot(h2, wd_ref[...], preferred_element_type=jnp.float32) + bd_ref[...]
    v_ref[...] = v
    y_ref[...] = x_ref[...] + v


_NC = 2
_NS = 16
_NW = _NC * _NS
_CG = 256
_G = _CG // 128
_CS = 128
_GS = _CS // 128
_S = 2


def _drain(dummy_hbm, dst, sem):
    pltpu.make_async_copy(dummy_hbm, dst, sem).wait()


def _sc_gather(f_hbm, jj_hbm, o_hbm,
               fbuf0, fbuf1, jbuf0, jbuf1, sg0, sg1, so0, so1, co=0):
    nch = o_hbm.shape[0] // _CG
    k_max = (nch + 2 * _NW - 1) // (2 * _NW) * 2
    wid = lax.axis_index("s") * _NC + lax.axis_index("c")
    bufs = ((fbuf0, jbuf0, sg0, so0), (fbuf1, jbuf1, sg1, so1))
    dummy = f_hbm.at[pl.ds(0, _CG)]

    def fire(k, b):
        fbuf, jbuf, sg, _ = bufs[b]
        ci = wid + k * _NW

        @pl.when((ci >= 0) & (ci < nch))
        def _():
            pltpu.sync_copy(jj_hbm.at[pl.ds((co + ci) * _G, _G)], jbuf)
            for g in range(_G):
                pltpu.async_copy(f_hbm.at[jbuf.at[g]],
                                 fbuf.at[pl.ds(g * 128, 128)], sg)

    def finish(k, b):
        fbuf, _, sg, so = bufs[b]
        ci = wid + k * _NW

        @pl.when((ci >= 0) & (ci < nch))
        def _():
            _drain(dummy, fbuf, sg)
            pltpu.async_copy(fbuf, o_hbm.at[pl.ds(ci * _CG, _CG)], so)

    def drain_out(k, b):
        fbuf, _, _, so = bufs[b]
        ci = wid + k * _NW

        @pl.when((ci >= 0) & (ci < nch))
        def _():
            _drain(dummy, fbuf, so)

    fire(0, 0)

    def body(k2, carry):
        for b in range(2):
            k = 2 * k2 + b
            nb = 1 - b
            drain_out(k - 1, nb)
            fire(k + 1, nb)
            finish(k, b)
        return carry

    lax.fori_loop(0, k_max // 2, body, 0)
    drain_out(k_max - 1, (k_max - 1) % 2)
    drain_out(k_max, k_max % 2)


def _sc_scatter(wf_hbm, ii_hbm, z_hbm, o_hbm,
                wbuf0, wbuf1, ibuf0, ibuf1, si0, si1, ss0, ss1, acc, co=0):
    n_pad = o_hbm.shape[1]
    e_rows = wf_hbm.shape[0]
    nch = e_rows // _CS
    k_max = (nch + 2 * _NW - 1) // (2 * _NW) * 2
    cid = lax.axis_index("c")
    sid = lax.axis_index("s")
    wid = sid * _NC + cid
    rows = n_pad // _NS
    bufs = ((wbuf0, ibuf0, si0, ss0), (wbuf1, ibuf1, si1, ss1))
    dummy = wf_hbm.at[pl.ds(0, _CS)]

    pltpu.sync_copy(z_hbm.at[pl.ds(sid * rows, rows)],
                    acc.at[pl.ds(sid * rows, rows)])
    plsc.subcore_barrier()

    def fire(k, b):
        wbuf, ibuf, si, _ = bufs[b]
        ci = wid + k * _NW

        @pl.when((ci >= 0) & (ci < nch))
        def _():
            pltpu.sync_copy(ii_hbm.at[pl.ds((co + ci) * _GS, _GS)], ibuf)
            pltpu.async_copy(wf_hbm.at[pl.ds(ci * _CS, _CS)], wbuf, si)

    def finish(k, b):
        wbuf, ibuf, si, ss = bufs[b]
        ci = wid + k * _NW

        @pl.when((ci >= 0) & (ci < nch))
        def _():
            _drain(dummy, wbuf, si)
            for g in range(_GS):
                pltpu.async_copy(wbuf.at[pl.ds(g * 128, 128)],
                                 acc.at[ibuf.at[g]], ss, add=True)

    def drain_sc(k, b):
        wbuf, _, _, ss = bufs[b]
        ci = wid + k * _NW

        @pl.when((ci >= 0) & (ci < nch))
        def _():
            _drain(dummy, wbuf, ss)

    fire(0, 0)

    def body(k2, carry):
        for b in range(2):
            k = 2 * k2 + b
            nb = 1 - b
            drain_sc(k - 1, nb)
            fire(k + 1, nb)
            finish(k, b)
        return carry

    lax.fori_loop(0, k_max // 2, body, 0)
    drain_sc(k_max - 1, (k_max - 1) % 2)
    drain_sc(k_max, k_max % 2)
    plsc.subcore_barrier()
    pltpu.sync_copy(acc.at[pl.ds(sid * rows, rows)],
                    o_hbm.at[cid, pl.ds(sid * rows, rows)])


def kernel(x, dijk, idx_j, seg_i, seg_j, W1, b1, W2, b2, Wf, bf, Wo, bo, Wd, bd):
    n_nodes, d = x.shape
    e_rows = dijk.shape[0]

    jj = jnp.take(idx_j, seg_j)
    ii = jnp.take(seg_i, seg_j)
    jj2 = jj.reshape(e_rows // 128, 128)
    ii2 = ii.reshape(e_rows // 128, 128)
    n_pad = ((n_nodes + (8 * _NS) - 1) // (8 * _NS)) * (8 * _NS)
    zeros = jnp.zeros((n_pad, d), jnp.float32)

    bn = 2000
    f = pl.pallas_call(
        _f_body,
        grid=(n_nodes // bn,),
        in_specs=[
            pl.BlockSpec((bn, d), lambda i: (i, 0)),
            pl.BlockSpec((d, d), lambda i: (0, 0)),
            pl.BlockSpec((1, d), lambda i: (0, 0)),
        ],
        out_specs=pl.BlockSpec((bn, d), lambda i: (i, 0)),
        out_shape=jax.ShapeDtypeStruct((n_nodes, d), jnp.float32),
    )(x, Wf, bf.reshape(1, d))

    mesh = plsc.VectorSubcoreMesh(core_axis_name="c", subcore_axis_name="s")
    es = e_rows // _S
    be = 2000
    gather_scratch = [
        pltpu.VMEM((_CG, d), jnp.float32),
        pltpu.VMEM((_CG, d), jnp.float32),
        pltpu.VMEM((_G, 128), jnp.int32),
        pltpu.VMEM((_G, 128), jnp.int32),
        pltpu.SemaphoreType.DMA,
        pltpu.SemaphoreType.DMA,
        pltpu.SemaphoreType.DMA,
        pltpu.SemaphoreType.DMA,
    ]
    scatter_scratch = [
        pltpu.VMEM((_CS, d), jnp.float32),
        pltpu.VMEM((_CS, d), jnp.float32),
        pltpu.VMEM((_GS, 128), jnp.int32),
        pltpu.VMEM((_GS, 128), jnp.int32),
        pltpu.SemaphoreType.DMA,
        pltpu.SemaphoreType.DMA,
        pltpu.SemaphoreType.DMA,
        pltpu.SemaphoreType.DMA,
        pltpu.VMEM_SHARED((n_pad, d), jnp.float32),
    ]

    fjjs = [
        pl.kernel(
            functools.partial(_sc_gather, co=s * (es // _CG)),
            out_type=jax.ShapeDtypeStruct((es, d), jnp.float32),
            mesh=mesh,
            scratch_types=gather_scratch,
        )(f, jj2)
        for s in range(_S)
    ]

    wfs = [
        pl.pallas_call(
            _edge_body,
            grid=(es // be,),
            in_specs=[
                pl.BlockSpec((be, d), lambda i, ro=s * (es // be): (i + ro, 0)),
                pl.BlockSpec((be, d), lambda i: (i, 0)),
                pl.BlockSpec((d, d), lambda i: (0, 0)),
                pl.BlockSpec((1, d), lambda i: (0, 0)),
                pl.BlockSpec((d, d), lambda i: (0, 0)),
                pl.BlockSpec((1, d), lambda i: (0, 0)),
            ],
            out_specs=pl.BlockSpec((be, d), lambda i: (i, 0)),
            out_shape=jax.ShapeDtypeStruct((es, d), jnp.float32),
        )(dijk, fjjs[s], W1, b1.reshape(1, d), W2, b2.reshape(1, d))
        for s in range(_S)
    ]

    convs = [
        pl.kernel(
            functools.partial(_sc_scatter, co=s * (es // _CS)),
            out_type=jax.ShapeDtypeStruct((_NC, n_pad, d), jnp.float32),
            mesh=mesh,
            scratch_types=scatter_scratch,
        )(wfs[s], ii2, zeros)
        for s in range(_S)
    ]

    y, v = pl.pallas_call(
        _out_body,
        grid=(n_nodes // bn,),
        in_specs=[
            pl.BlockSpec((_NC, bn, d), lambda i: (0, i, 0))
            for _ in range(_S)
        ] + [
            pl.BlockSpec((bn, d), lambda i: (i, 0)),
            pl.BlockSpec((d, d), lambda i: (0, 0)),
            pl.BlockSpec((1, d), lambda i: (0, 0)),
            pl.BlockSpec((d, d), lambda i: (0, 0)),
            pl.BlockSpec((1, d), lambda i: (0, 0)),
        ],
        out_specs=[
            pl.BlockSpec((bn, d), lambda i: (i, 0)),
            pl.BlockSpec((bn, d), lambda i: (i, 0)),
        ],
        out_shape=[
            jax.ShapeDtypeStruct((n_nodes, d), jnp.float32),
            jax.ShapeDtypeStruct((n_nodes, d), jnp.float32),
        ],
    )(*convs, x, Wo, bo.reshape(1, d), Wd, bd.reshape(1, d))
    return (y, v)
